# baseline (speedup 1.0000x reference)
import jax
import jax.numpy as jnp
from jax import lax
from jax.experimental import pallas as pl
from jax.experimental.pallas import tpu as pltpu
from jax.experimental.pallas import tpu_sc as plsc

NC, NS = 2, 16
NW = NC * NS
BATCH, SEQ, HID = 4, 8192, 2048
ROWS = BATCH * SEQ
RPW = ROWS // NW
T = 16
NCHUNK = RPW // T


def _sc_embed(idx_hbm, emb_hbm, pos_hbm, out_hbm, idx_v, buf, gbuf, sem):
    wid = lax.axis_index("s") * NC + lax.axis_index("c")
    base = pl.multiple_of(wid * RPW, RPW)

    def chunk(c, carry):
        r0 = pl.multiple_of(base + c * T, T)
        p0 = pl.multiple_of(lax.rem(base, SEQ) + c * T, T)
        pltpu.sync_copy(idx_hbm.at[pl.ds(r0, T)], idx_v)
        gather = pltpu.async_copy(emb_hbm.at[idx_v], gbuf, sem)
        pltpu.sync_copy(pos_hbm.at[pl.ds(p0, T)], buf)
        gather.wait()

        @plsc.parallel_loop(0, HID, step=16)
        def add_body(j):
            for t in range(T):
                plsc.addupdate(buf.at[t, pl.ds(j, 16)], gbuf[t, pl.ds(j, 16)])

        pltpu.sync_copy(buf, out_hbm.at[pl.ds(r0, T)])
        return carry

    lax.fori_loop(0, NCHUNK, chunk, 0)


def kernel(input_bytes, emb_table, pos_table):
    idx = input_bytes.reshape(ROWS).astype(jnp.int32)
    mesh = plsc.VectorSubcoreMesh(
        core_axis_name="c", subcore_axis_name="s",
        num_cores=NC, num_subcores=NS)
    out = pl.kernel(
        _sc_embed,
        out_type=jax.ShapeDtypeStruct((ROWS, HID), jnp.float32),
        mesh=mesh,
        scratch_types=[
            pltpu.VMEM((T,), jnp.int32),
            pltpu.VMEM((T, HID), jnp.float32),
            pltpu.VMEM((T, HID), jnp.float32),
            pltpu.SemaphoreType.DMA,
        ],
    )(idx, emb_table, pos_table)
    return out.reshape(BATCH, SEQ, HID)


# speedup vs baseline: 1.5903x; 1.5903x over previous
import jax
import jax.numpy as jnp
from jax import lax
from jax.experimental import pallas as pl
from jax.experimental.pallas import tpu as pltpu
from jax.experimental.pallas import tpu_sc as plsc

NC, NS = 2, 16
NW = NC * NS
BATCH, SEQ, HID = 4, 8192, 2048
ROWS = BATCH * SEQ
RPW = ROWS // NW
T = 8
NCHUNK = RPW // T


def _sc_embed(idx_hbm, emb_hbm, pos_hbm, out_hbm,
              idx_v, buf0, buf1, gbuf0, gbuf1,
              gsem0, gsem1, psem0, psem1, osem0, osem1):
    wid = lax.axis_index("s") * NC + lax.axis_index("c")
    base = pl.multiple_of(wid * RPW, RPW)
    pbase = lax.rem(base, SEQ)
    bufs = (buf0, buf1)
    gbufs = (gbuf0, gbuf1)
    gsems = (gsem0, gsem1)
    psems = (psem0, psem1)
    osems = (osem0, osem1)

    pltpu.sync_copy(idx_hbm.at[pl.ds(base, RPW)], idx_v)

    def launch(c, s):
        r0 = pl.multiple_of(base + c * T, T)
        p0 = pl.multiple_of(pbase + c * T, T)

        @pl.when(c >= 2)
        def _():
            rp = pl.multiple_of(base + (c - 2) * T, T)
            pltpu.make_async_copy(bufs[s], out_hbm.at[pl.ds(rp, T)],
                                  osems[s]).wait()

        pltpu.async_copy(emb_hbm.at[idx_v.at[pl.ds(c * T, T)]],
                         gbufs[s], gsems[s])
        pltpu.async_copy(pos_hbm.at[pl.ds(p0, T)], bufs[s], psems[s])

    def finish(c, s):
        r0 = pl.multiple_of(base + c * T, T)
        p0 = pl.multiple_of(pbase + c * T, T)
        pltpu.make_async_copy(emb_hbm.at[idx_v.at[pl.ds(c * T, T)]],
                              gbufs[s], gsems[s]).wait()
        pltpu.make_async_copy(pos_hbm.at[pl.ds(p0, T)], bufs[s],
                              psems[s]).wait()

        @plsc.parallel_loop(0, HID, step=16)
        def add_body(j):
            for t in range(T):
                plsc.addupdate(bufs[s].at[t, pl.ds(j, 16)],
                               gbufs[s][t, pl.ds(j, 16)])

        pltpu.async_copy(bufs[s], out_hbm.at[pl.ds(r0, T)], osems[s])

    launch(0, 0)

    def body(i, carry):
        c = 2 * i
        launch(c + 1, 1)
        finish(c, 0)
        launch(c + 2, 0)
        finish(c + 1, 1)
        return carry

    lax.fori_loop(0, NCHUNK // 2 - 1, body, 0)

    c = NCHUNK - 2
    launch(c + 1, 1)
    finish(c, 0)
    finish(c + 1, 1)
    pltpu.make_async_copy(bufs[0], out_hbm.at[pl.ds(base + c * T, T)],
                          osems[0]).wait()
    pltpu.make_async_copy(bufs[1], out_hbm.at[pl.ds(base + (c + 1) * T, T)],
                          osems[1]).wait()


def kernel(input_bytes, emb_table, pos_table):
    idx = input_bytes.reshape(ROWS).astype(jnp.int32)
    mesh = plsc.VectorSubcoreMesh(
        core_axis_name="c", subcore_axis_name="s",
        num_cores=NC, num_subcores=NS)
    out = pl.kernel(
        _sc_embed,
        out_type=jax.ShapeDtypeStruct((ROWS, HID), jnp.float32),
        mesh=mesh,
        scratch_types=[
            pltpu.VMEM((RPW,), jnp.int32),
            pltpu.VMEM((T, HID), jnp.float32),
            pltpu.VMEM((T, HID), jnp.float32),
            pltpu.VMEM((T, HID), jnp.float32),
            pltpu.VMEM((T, HID), jnp.float32),
            pltpu.SemaphoreType.DMA,
            pltpu.SemaphoreType.DMA,
            pltpu.SemaphoreType.DMA,
            pltpu.SemaphoreType.DMA,
            pltpu.SemaphoreType.DMA,
            pltpu.SemaphoreType.DMA,
        ],
    )(idx, emb_table, pos_table)
    return out.reshape(BATCH, SEQ, HID)
